# Initial kernel scaffold; baseline (speedup 1.0000x reference)
import jax
import jax.numpy as jnp
from jax.experimental import pallas as pl


def kernel(x, edge_index, edge_type, W1, root1, b1, W2, root2, b2):
    raise NotImplementedError("write your pallas kernel here")


# speedup vs baseline: 6.2436x; 6.2436x over previous
import functools

import jax
import jax.numpy as jnp
from jax import lax
from jax.experimental import pallas as pl
from jax.experimental.pallas import tpu as pltpu
from jax.experimental.pallas import tpu_sc as plsc

N_NODES = 10000
N_PAD = 10240
N_EDGES = 160000
N_REL = 4
KEYS_PAD = 40960
EDGES_PER_TILE = N_EDGES // 16
CHUNK = 400


def _sc_mesh():
    return plsc.VectorSubcoreMesh(core_axis_name="c", subcore_axis_name="s")


@functools.partial(
    pl.kernel,
    mesh=_sc_mesh(),
    compiler_params=pltpu.CompilerParams(needs_layout_passes=False, use_tc_tiling_on_sc=False),
    out_type=jax.ShapeDtypeStruct((N_EDGES,), jnp.float32),
    scratch_types=[
        pltpu.VMEM_SHARED((KEYS_PAD,), jnp.float32),
        pltpu.VMEM((KEYS_PAD,), jnp.float32),
        pltpu.VMEM((CHUNK,), jnp.int32),
        pltpu.VMEM((CHUNK,), jnp.int32),
        pltpu.VMEM((CHUNK,), jnp.int32),
        pltpu.VMEM((CHUNK,), jnp.float32),
        pltpu.VMEM((CHUNK,), jnp.float32),
    ],
)
def _scale_kernel(dst_h, ety_h, scale_h, cnt, invv, dstb, etyb, keyb,
                  onesb, scaleb):
    c = lax.axis_index("c")
    s = lax.axis_index("s")
    spt = KEYS_PAD // 16

    def z16(i, carry):
        invv[pl.ds(i * 16, 16)] = jnp.zeros((16,), jnp.float32)
        return carry

    lax.fori_loop(0, spt // 16, z16, 0)
    pltpu.sync_copy(invv.at[pl.ds(0, spt)], cnt.at[pl.ds(s * spt, spt)])

    def o16(i, carry):
        onesb[pl.ds(i * 16, 16)] = jnp.ones((16,), jnp.float32)
        return carry

    lax.fori_loop(0, CHUNK // 16, o16, 0)
    plsc.subcore_barrier()

    def chunk(k, carry):
        base = s * EDGES_PER_TILE + k * CHUNK
        pltpu.sync_copy(dst_h.at[pl.ds(base, CHUNK)], dstb)
        pltpu.sync_copy(ety_h.at[pl.ds(base, CHUNK)], etyb)

        def g16(i, cc):
            sl = pl.ds(i * 16, 16)
            keyb[sl] = dstb[sl] * N_REL + etyb[sl]
            return cc

        lax.fori_loop(0, CHUNK // 16, g16, 0)
        pltpu.sync_copy(onesb, cnt.at[keyb], add=True)
        return carry

    lax.fori_loop(0, EDGES_PER_TILE // CHUNK, chunk, 0)
    plsc.subcore_barrier()

    @pl.when(c == 0)
    def _():
        pltpu.sync_copy(cnt, invv)

        def inv16(i, carry):
            sl = pl.ds(i * 16, 16)
            invv[sl] = 1.0 / jnp.maximum(invv[sl], 1.0)
            return carry

        lax.fori_loop(0, KEYS_PAD // 16, inv16, 0)

        def chunk2(k, carry):
            base = s * EDGES_PER_TILE + k * CHUNK
            pltpu.sync_copy(dst_h.at[pl.ds(base, CHUNK)], dstb)
            pltpu.sync_copy(ety_h.at[pl.ds(base, CHUNK)], etyb)

            def g16(i, cc):
                sl = pl.ds(i * 16, 16)
                key16 = dstb[sl] * N_REL + etyb[sl]
                scaleb[sl] = plsc.load_gather(invv, [key16])
                return cc

            lax.fori_loop(0, CHUNK // 16, g16, 0)
            pltpu.sync_copy(scaleb, scale_h.at[pl.ds(base, CHUNK)])
            return carry

        lax.fori_loop(0, EDGES_PER_TILE // CHUNK, chunk2, 0)


def _make_edge_pass(dh, ck):
    @functools.partial(
        pl.kernel,
        mesh=_sc_mesh(),
        compiler_params=pltpu.CompilerParams(needs_layout_passes=False, use_tc_tiling_on_sc=False),
        out_type=jax.ShapeDtypeStruct((2, N_PAD, dh), jnp.float32),
        scratch_types=[
            pltpu.VMEM_SHARED((N_PAD, dh), jnp.float32),
            pltpu.VMEM((ck,), jnp.int32),
            pltpu.VMEM((ck,), jnp.int32),
            pltpu.VMEM((ck,), jnp.int32),
            pltpu.VMEM((ck,), jnp.int32),
            pltpu.VMEM((ck * 16,), jnp.float32),
            pltpu.VMEM((ck, dh), jnp.float32),
            pltpu.VMEM((16,), jnp.float32),
            pltpu.SemaphoreType.DMA,
        ],
    )
    def edge_pass(taba_h, tabb_h, inita_h, initb_h, src_h, ety_h, dst_h,
                  scale_h, out_h, acc, srcb, etyb, dstb, gidxb, scaleb,
                  rows, scb, sem):
        c = lax.axis_index("c")
        s = lax.axis_index("s")
        rpt = N_PAD // 16

        @pl.when(c == 0)
        def _():
            pltpu.sync_copy(inita_h.at[pl.ds(s * rpt, rpt)],
                            acc.at[pl.ds(s * rpt, rpt)])

        @pl.when(c == 1)
        def _():
            pltpu.sync_copy(initb_h.at[pl.ds(s * rpt, rpt)],
                            acc.at[pl.ds(s * rpt, rpt)])

        plsc.subcore_barrier()

        def chunk(k, carry):
            base = s * EDGES_PER_TILE + k * ck
            pltpu.sync_copy(src_h.at[pl.ds(base, ck)], srcb)
            pltpu.sync_copy(ety_h.at[pl.ds(base, ck)], etyb)
            pltpu.sync_copy(dst_h.at[pl.ds(base, ck)], dstb)
            pltpu.sync_copy(scale_h.at[pl.ds(base * 16, ck * 16)], scaleb)

            def g16(i, cc):
                sl = pl.ds(i * 16, 16)
                gidxb[sl] = srcb[sl] * N_REL + etyb[sl]
                return cc

            lax.fori_loop(0, ck // 16, g16, 0)

            @pl.when(c == 0)
            def _():
                pltpu.async_copy(taba_h.at[gidxb], rows, sem).wait()

            @pl.when(c == 1)
            def _():
                pltpu.async_copy(tabb_h.at[gidxb], rows, sem).wait()

            for e in range(ck):
                spl = scaleb[pl.ds(e * 16, 16)]
                for q in range(dh // 16):
                    qs = pl.ds(q * 16, 16)
                    rows[e, qs] = rows[e, qs] * spl
            pltpu.sync_copy(rows, acc.at[dstb], add=True)
            return carry

        lax.fori_loop(0, EDGES_PER_TILE // ck, chunk, 0)
        plsc.subcore_barrier()
        pltpu.sync_copy(acc.at[pl.ds(s * rpt, rpt)],
                        out_h.at[c, pl.ds(s * rpt, rpt)])

    return edge_pass


_edge_pass_l1 = _make_edge_pass(128, 80)
_edge_pass_l2 = _make_edge_pass(32, 400)


def _make_dense(din, dout, do_relu):
    dhi = din // 2
    dho = dout // 2
    blk = 640
    nblk = N_PAD // blk

    def body(ha_ref, hb_ref, w_ref, root_ref, b_ref,
             taba_ref, tabb_ref, oa_ref, ob_ref):
        h = jnp.concatenate([ha_ref[...], hb_ref[...]], axis=1)
        if do_relu:
            h = jnp.maximum(h, 0.0)
        o = jnp.dot(h, root_ref[...],
                    preferred_element_type=jnp.float32) + b_ref[...]
        oa_ref[...] = o[:, :dho]
        ob_ref[...] = o[:, dho:]
        for r in range(N_REL):
            hr = jnp.dot(h, w_ref[r], preferred_element_type=jnp.float32)
            taba_ref[:, r, :] = hr[:, :dho]
            tabb_ref[:, r, :] = hr[:, dho:]

    return pl.pallas_call(
        body,
        grid=(nblk,),
        in_specs=[
            pl.BlockSpec((blk, dhi), lambda i: (i, 0)),
            pl.BlockSpec((blk, dhi), lambda i: (i, 0)),
            pl.BlockSpec((N_REL, din, dout), lambda i: (0, 0, 0)),
            pl.BlockSpec((din, dout), lambda i: (0, 0)),
            pl.BlockSpec((1, dout), lambda i: (0, 0)),
        ],
        out_specs=[
            pl.BlockSpec((blk, N_REL, dho), lambda i: (i, 0, 0)),
            pl.BlockSpec((blk, N_REL, dho), lambda i: (i, 0, 0)),
            pl.BlockSpec((blk, dho), lambda i: (i, 0)),
            pl.BlockSpec((blk, dho), lambda i: (i, 0)),
        ],
        out_shape=[
            jax.ShapeDtypeStruct((N_PAD, N_REL, dho), jnp.float32),
            jax.ShapeDtypeStruct((N_PAD, N_REL, dho), jnp.float32),
            jax.ShapeDtypeStruct((N_PAD, dho), jnp.float32),
            jax.ShapeDtypeStruct((N_PAD, dho), jnp.float32),
        ],
    )


_dense_l1 = _make_dense(256, 256, do_relu=False)
_dense_l2 = _make_dense(256, 64, do_relu=True)


def kernel(x, edge_index, edge_type, W1, root1, b1, W2, root2, b2):
    src = edge_index[0]
    dst = edge_index[1]
    ety = edge_type

    scale = _scale_kernel(dst, ety)
    scale16 = jnp.broadcast_to(scale[:, None],
                               (N_EDGES, 16)).reshape(N_EDGES * 16)

    xp = jnp.pad(x, ((0, N_PAD - N_NODES), (0, 0)))
    xa = xp[:, :128]
    xb = xp[:, 128:]
    t1a, t1b, o1a, o1b = _dense_l1(xa, xb, W1, root1, b1[None, :])
    y1 = _edge_pass_l1(t1a.reshape(N_PAD * N_REL, 128),
                       t1b.reshape(N_PAD * N_REL, 128),
                       o1a, o1b, src, ety, dst, scale16)

    t2a, t2b, o2a, o2b = _dense_l2(y1[0], y1[1], W2, root2, b2[None, :])
    y2 = _edge_pass_l2(t2a.reshape(N_PAD * N_REL, 32),
                       t2b.reshape(N_PAD * N_REL, 32),
                       o2a, o2b, src, ety, dst, scale16)
    return jnp.concatenate([y2[0, :N_NODES], y2[1, :N_NODES]], axis=1)
